# baseline (speedup 1.0000x reference)
import jax
import jax.numpy as jnp
from jax.experimental import pallas as pl
from jax.experimental.pallas import tpu as pltpu


def _fused_kernel(x_ref, w_ref, b_ref, an_top_ref, an_bot_ref,
                  o_top_ref, o_bot_ref):
    x = x_ref[...]
    w = w_ref[...]
    b = b_ref[...]
    t0 = jnp.dot(an_top_ref[...], x, preferred_element_type=jnp.float32)
    o_top_ref[...] = jax.nn.sigmoid(
        jnp.dot(t0, w, preferred_element_type=jnp.float32) + b)
    t1 = jnp.dot(an_bot_ref[...], x, preferred_element_type=jnp.float32)
    o_bot_ref[...] = jax.nn.sigmoid(
        jnp.dot(t1, w, preferred_element_type=jnp.float32) + b)


def kernel(An, X, weight, bias):
    n, f = X.shape
    u = weight.shape[1]
    bm = 200
    half = n // 2
    steps = half // bm

    o_top, o_bot = pl.pallas_call(
        _fused_kernel,
        grid=(steps,),
        in_specs=[
            pl.BlockSpec((n, f), lambda i: (0, 0)),
            pl.BlockSpec((f, u), lambda i: (0, 0)),
            pl.BlockSpec((1, u), lambda i: (0, 0)),
            pl.BlockSpec((bm, n), lambda i: (i, 0)),
            pl.BlockSpec((bm, n), lambda i: (i + half // bm, 0)),
        ],
        out_specs=[
            pl.BlockSpec((bm, u), lambda i: (i, 0)),
            pl.BlockSpec((bm, u), lambda i: (i, 0)),
        ],
        out_shape=[
            jax.ShapeDtypeStruct((half, u), jnp.float32),
            jax.ShapeDtypeStruct((half, u), jnp.float32),
        ],
        compiler_params=pltpu.CompilerParams(
            dimension_semantics=("parallel",),
        ),
    )(X, weight, bias.reshape(1, u), An, An)
    return jnp.concatenate([o_top, o_bot], axis=0)


# speedup vs baseline: 1.1108x; 1.1108x over previous
import functools

import jax
import jax.numpy as jnp
from jax.experimental import pallas as pl
from jax.experimental.pallas import tpu as pltpu


def _fused_kernel(x_ref, w_ref, b_ref, an_ref, o_ref, an_buf, sem, *, bm, nbuf):
    n = x_ref.shape[0]
    steps = n // bm

    def start_copy(step):
        slot = jax.lax.rem(step, nbuf)
        pltpu.make_async_copy(
            an_ref.at[pl.ds(step * bm, bm), :],
            an_buf.at[slot],
            sem.at[slot],
        ).start()

    for s in range(nbuf):
        start_copy(s)

    x = x_ref[...]
    w = w_ref[...]
    b = b_ref[...]

    def body(i, carry):
        slot = jax.lax.rem(i, nbuf)
        pltpu.make_async_copy(
            an_ref.at[pl.ds(i * bm, bm), :],
            an_buf.at[slot],
            sem.at[slot],
        ).wait()
        t = jnp.dot(an_buf[slot], x, preferred_element_type=jnp.float32)
        z = jnp.dot(t, w, preferred_element_type=jnp.float32)
        o_ref[pl.ds(i * bm, bm), :] = jax.nn.sigmoid(z + b)

        @pl.when(i + nbuf < steps)
        def _():
            start_copy(i + nbuf)

        return carry

    jax.lax.fori_loop(0, steps, body, 0)


def kernel(An, X, weight, bias):
    n, f = X.shape
    u = weight.shape[1]
    bm = 200
    nbuf = 5

    return pl.pallas_call(
        functools.partial(_fused_kernel, bm=bm, nbuf=nbuf),
        in_specs=[
            pl.BlockSpec(memory_space=pltpu.VMEM),
            pl.BlockSpec(memory_space=pltpu.VMEM),
            pl.BlockSpec(memory_space=pltpu.VMEM),
            pl.BlockSpec(memory_space=pltpu.HBM),
        ],
        out_specs=pl.BlockSpec(memory_space=pltpu.VMEM),
        out_shape=jax.ShapeDtypeStruct((n, u), jnp.float32),
        scratch_shapes=[
            pltpu.VMEM((nbuf, bm, n), jnp.float32),
            pltpu.SemaphoreType.DMA((nbuf,)),
        ],
    )(X, weight, bias.reshape(1, u), An)


# speedup vs baseline: 1.1485x; 1.0339x over previous
import jax
import jax.numpy as jnp
from jax.experimental import pallas as pl
from jax.experimental.pallas import tpu as pltpu


def _fused_kernel(x_ref, w_ref, b_ref, an_ref, o_ref):
    t = jnp.dot(an_ref[...], x_ref[...], preferred_element_type=jnp.float32)
    z = jnp.dot(t, w_ref[...], preferred_element_type=jnp.float32)
    o_ref[...] = jax.nn.sigmoid(z + b_ref[...])


def kernel(An, X, weight, bias):
    n, f = X.shape
    u = weight.shape[1]
    bm = 400

    return pl.pallas_call(
        _fused_kernel,
        grid=(n // bm,),
        in_specs=[
            pl.BlockSpec((n, f), lambda i: (0, 0)),
            pl.BlockSpec((f, u), lambda i: (0, 0)),
            pl.BlockSpec((1, u), lambda i: (0, 0)),
            pl.BlockSpec((bm, n), lambda i: (i, 0)),
        ],
        out_specs=pl.BlockSpec((bm, u), lambda i: (i, 0)),
        out_shape=jax.ShapeDtypeStruct((n, u), jnp.float32),
        compiler_params=pltpu.CompilerParams(
            dimension_semantics=("parallel",),
        ),
    )(X, weight, bias.reshape(1, u), An)
